# baseline (speedup 1.0000x reference)
import functools
import math

import ml_dtypes
import numpy as np
import jax
import jax.numpy as jnp
from jax.experimental import pallas as pl
from jax.experimental.pallas import tpu as pltpu

_NUM_ANGLE = 100
_NUM_RHO = 100
_COLS_PER_STEP = 2560


@functools.lru_cache(maxsize=None)
def _rho_table(H, W, num_angle, num_rho):
    irho = int(math.sqrt(H * H + W * W) + 1) / float(num_rho)
    itheta = math.pi / num_angle
    angles = np.arange(num_angle, dtype=np.float64) * itheta
    cosv = (np.cos(angles) / irho).astype(np.float32)
    sinv = (np.sin(angles) / irho).astype(np.float32)
    ys, xs = np.meshgrid(np.arange(H), np.arange(W), indexing='ij')
    xx = (xs - W // 2).reshape(-1).astype(np.float32)
    yy = (ys - H // 2).reshape(-1).astype(np.float32)
    r = np.round(xx[None, :] * cosv[:, None] + yy[None, :] * sinv[:, None])
    r = r.astype(np.int32) + num_rho // 2
    r = np.clip(r, 0, num_rho - 1)
    return r


@functools.lru_cache(maxsize=None)
def _vote_matrix(H, W, num_angle, num_rho, cols_pad):
    ridx = _rho_table(H, W, num_angle, num_rho)
    HW = H * W
    flat = ridx + (np.arange(num_angle, dtype=np.int32) * num_rho)[:, None]
    n = np.zeros((HW, cols_pad), dtype=ml_dtypes.float8_e4m3fn)
    n[np.arange(HW)[None, :], flat] = 1
    return n


def _dht_body(n_ref, xf_ref, out_ref):
    n = n_ref[...].astype(jnp.bfloat16)
    out_ref[...] = jnp.dot(xf_ref[...], n, preferred_element_type=jnp.float32)


def kernel(x):
    B, C, H, W = x.shape
    HW = H * W
    BC = B * C
    AR = _NUM_ANGLE * _NUM_RHO
    cols_pad = ((AR + _COLS_PER_STEP - 1) // _COLS_PER_STEP) * _COLS_PER_STEP
    nsteps = cols_pad // _COLS_PER_STEP

    nmat = jnp.asarray(_vote_matrix(H, W, _NUM_ANGLE, _NUM_RHO, cols_pad))
    xf = x.reshape(BC, HW).astype(jnp.bfloat16)

    out = pl.pallas_call(
        _dht_body,
        grid=(nsteps,),
        in_specs=[
            pl.BlockSpec((HW, _COLS_PER_STEP), lambda i: (0, i)),
            pl.BlockSpec((BC, HW), lambda i: (0, 0)),
        ],
        out_specs=pl.BlockSpec((BC, _COLS_PER_STEP), lambda i: (0, i)),
        out_shape=jax.ShapeDtypeStruct((BC, cols_pad), jnp.float32),
        compiler_params=pltpu.CompilerParams(
            vmem_limit_bytes=100 * 1024 * 1024),
    )(nmat, xf)

    return out[:, :AR].reshape(B, C, _NUM_ANGLE, _NUM_RHO)


# speedup vs baseline: 1.2604x; 1.2604x over previous
import functools
import math

import ml_dtypes
import numpy as np
import jax
import jax.numpy as jnp
from jax.experimental import pallas as pl
from jax.experimental.pallas import tpu as pltpu

_NUM_ANGLE = 100
_NUM_RHO = 100
_COLS_PER_STEP = 1280


@functools.lru_cache(maxsize=None)
def _rho_table(H, W, num_angle, num_rho):
    irho = int(math.sqrt(H * H + W * W) + 1) / float(num_rho)
    itheta = math.pi / num_angle
    angles = np.arange(num_angle, dtype=np.float64) * itheta
    cosv = (np.cos(angles) / irho).astype(np.float32)
    sinv = (np.sin(angles) / irho).astype(np.float32)
    ys, xs = np.meshgrid(np.arange(H), np.arange(W), indexing='ij')
    xx = (xs - W // 2).reshape(-1).astype(np.float32)
    yy = (ys - H // 2).reshape(-1).astype(np.float32)
    r = np.round(xx[None, :] * cosv[:, None] + yy[None, :] * sinv[:, None])
    r = r.astype(np.int32) + num_rho // 2
    r = np.clip(r, 0, num_rho - 1)
    return r


@functools.lru_cache(maxsize=None)
def _vote_matrix(H, W, num_angle, num_rho, cols_pad):
    ridx = _rho_table(H, W, num_angle, num_rho)
    HW = H * W
    flat = ridx + (np.arange(num_angle, dtype=np.int32) * num_rho)[:, None]
    n = np.zeros((HW, cols_pad), dtype=ml_dtypes.float8_e4m3fn)
    n[np.arange(HW)[None, :], flat] = 1
    return n


def _dht_body(n_ref, xf_ref, out_ref):
    n = n_ref[...].astype(jnp.bfloat16)
    out_ref[...] = jnp.dot(xf_ref[...], n, preferred_element_type=jnp.float32)


def kernel(x):
    B, C, H, W = x.shape
    HW = H * W
    BC = B * C
    AR = _NUM_ANGLE * _NUM_RHO
    cols_pad = ((AR + _COLS_PER_STEP - 1) // _COLS_PER_STEP) * _COLS_PER_STEP
    nsteps = cols_pad // _COLS_PER_STEP

    nmat = jnp.asarray(_vote_matrix(H, W, _NUM_ANGLE, _NUM_RHO, cols_pad))
    xf = x.reshape(BC, HW).astype(jnp.bfloat16)

    out = pl.pallas_call(
        _dht_body,
        grid=(nsteps,),
        in_specs=[
            pl.BlockSpec((HW, _COLS_PER_STEP), lambda i: (0, i)),
            pl.BlockSpec((BC, HW), lambda i: (0, 0)),
        ],
        out_specs=pl.BlockSpec((BC, _COLS_PER_STEP), lambda i: (0, i)),
        out_shape=jax.ShapeDtypeStruct((BC, AR), jnp.float32),
    )(nmat, xf)

    return out.reshape(B, C, _NUM_ANGLE, _NUM_RHO)
